# baseline (speedup 1.0000x reference)
import math

import jax
import jax.numpy as jnp
from jax.experimental import pallas as pl

CAPACITY_FACTOR = 1.25


def kernel(hidden, Wr, W1, W2):
    Bb, Tt, Dd = hidden.shape
    N = Bb * Tt
    n_experts = Wr.shape[0]
    hidden_flat = hidden.reshape(N, Dd)
    router_logits = hidden_flat @ Wr.T
    router_probs = jax.nn.softmax(router_logits, axis=-1)
    capacity = min(int(math.ceil(CAPACITY_FACTOR * N / n_experts)), N)
    scores = router_probs.T
    weights, indices = jax.lax.top_k(scores, capacity)
    output = jnp.zeros_like(hidden_flat)
    for e in range(n_experts):
        expert_indices = indices[e]
        expert_weights = weights[e]
        expert_input = jnp.take(hidden_flat, expert_indices, axis=0)
        mid = jax.nn.gelu(
            jnp.dot(expert_input.astype(jnp.bfloat16), W1[e].T.astype(jnp.bfloat16),
                    preferred_element_type=jnp.float32),
            approximate=False)
        expert_out = jnp.dot(mid.astype(jnp.bfloat16), W2[e].T.astype(jnp.bfloat16),
                             preferred_element_type=jnp.float32)
        output = output.at[expert_indices].add(expert_weights[:, None] * expert_out)
    output = output.reshape(Bb, Tt, Dd)
    eps = 1e-08
    entropy = -(router_probs * jnp.log(router_probs + eps)).sum(axis=-1)
    aux_loss = -entropy.mean()
    return (output, aux_loss)


# speedup vs baseline: 2.6146x; 2.6146x over previous
import functools
import math

import jax
import jax.numpy as jnp
from jax import lax
from jax.experimental import pallas as pl
from jax.experimental.pallas import tpu as pltpu
from jax.experimental.pallas import tpu_sc as plsc

B, T, D, E, DFF = 2, 2048, 2048, 8, 8192
N = B * T
CAP = min(int(math.ceil(1.25 * N / E)), N)
CP = 672
TB = 512
BF = 1024
KF = DFF // BF
NC, NS, LANES = 2, 16, 16
CHUNK = 512
NCHUNK = N // CHUNK
EPW = (E * CP) // NS
NBAT = (EPW + LANES - 1) // LANES


def _router_body(x_ref, wr_ref, pt_ref, ent_ref):
    x = x_ref[...].astype(jnp.bfloat16)
    wr = wr_ref[...].astype(jnp.bfloat16)
    lt = lax.dot_general(wr, x, (((1,), (1,)), ((), ())),
                         preferred_element_type=jnp.float32)
    m = jnp.max(lt, axis=0, keepdims=True)
    p = jnp.exp(lt - m)
    p = p / jnp.sum(p, axis=0, keepdims=True)
    pt_ref[...] = p
    ent = jnp.sum(p * jnp.log(p + 1e-8))
    ent_ref[...] = ent.reshape(1, 1, 1)


def _router(hidden_flat, Wr):
    return pl.pallas_call(
        _router_body,
        grid=(N // TB,),
        in_specs=[
            pl.BlockSpec((TB, D), lambda i: (i, 0)),
            pl.BlockSpec((E, D), lambda i: (0, 0)),
        ],
        out_specs=[
            pl.BlockSpec((E, TB), lambda i: (0, i)),
            pl.BlockSpec((1, 1, 1), lambda i: (i, 0, 0)),
        ],
        out_shape=[
            jax.ShapeDtypeStruct((E, N), jnp.float32),
            jax.ShapeDtypeStruct((N // TB, 1, 1), jnp.float32),
        ],
        compiler_params=pltpu.CompilerParams(
            dimension_semantics=("arbitrary",)),
    )(hidden_flat, Wr)


_SC_MESH = plsc.VectorSubcoreMesh(
    core_axis_name="c", subcore_axis_name="s", num_cores=NC, num_subcores=NS)

_NV = N // LANES


def _topk_kernel(pt_hbm, idx_hbm, w_hbm, sco_v, bits_v, idx_v, w_v):
    cid = lax.axis_index("c")
    sid = lax.axis_index("s")
    wid = sid * NC + cid

    @pl.when(wid < E)
    def _():
        pltpu.sync_copy(pt_hbm.at[pl.ds(wid * N, N)], sco_v)
        zi = jnp.zeros((LANES,), jnp.int32)
        zf = jnp.zeros((LANES,), jnp.float32)

        def _tobits(i, _):
            bits_v[pl.ds(i * LANES, LANES)] = plsc.bitcast(
                sco_v[pl.ds(i * LANES, LANES)], jnp.int32)
            return 0
        lax.fori_loop(0, _NV, _tobits, 0)

        def _count_gt(t):
            def body(i, acc):
                v = bits_v[pl.ds(i * LANES, LANES)]
                return acc + jnp.where(v > t, 1, 0).astype(jnp.int32)
            acc = lax.fori_loop(0, _NV, body, jnp.zeros((LANES,), jnp.int32))
            return jnp.sum(acc)

        def bstep(_, carry):
            lo, hi, cnt_hi = carry
            mid = lax.shift_right_arithmetic(lo + hi, 1)
            c = _count_gt(mid)
            big = c >= CAP
            lo = jnp.where(big, mid, lo)
            hi = jnp.where(big, hi, mid)
            cnt_hi = jnp.where(big, cnt_hi, c)
            return lo, hi, cnt_hi
        _, kth, cnt_gt = lax.fori_loop(
            0, 31, bstep,
            (jnp.int32(-1), jnp.int32(0x40000000), jnp.int32(0)))
        need_eq = CAP - cnt_gt

        def _zfill(i, _):
            idx_v[pl.ds(i * LANES, LANES)] = zi
            w_v[pl.ds(i * LANES, LANES)] = zf
            return 0
        lax.fori_loop(0, (CP + LANES) // LANES, _zfill, 0)

        lane_iota = jnp.arange(LANES, dtype=jnp.int32)

        def cstep(i, carry):
            off, eq_seen = carry
            v = bits_v[pl.ds(i * LANES, LANES)]
            f = sco_v[pl.ds(i * LANES, LANES)]
            m_gt = v > kth
            m_eq = v == kth
            pc = plsc.cumsum(m_eq.astype(jnp.int32))
            sel = m_gt | (m_eq & ((eq_seen + pc) <= need_eq))
            toks = lane_iota + i * LANES
            plsc.store_compressed(idx_v.at[pl.ds(off, LANES)], toks, mask=sel)
            plsc.store_compressed(w_v.at[pl.ds(off, LANES)], f, mask=sel)
            off = off + jnp.sum(sel.astype(jnp.int32))
            eq_seen = eq_seen + jnp.sum(m_eq.astype(jnp.int32))
            return off, eq_seen
        lax.fori_loop(0, _NV, cstep, (jnp.int32(0), jnp.int32(0)))

        pltpu.sync_copy(idx_v.at[pl.ds(0, CP)], idx_hbm.at[pl.ds(wid * CP, CP)])
        pltpu.sync_copy(w_v.at[pl.ds(0, CP)], w_hbm.at[pl.ds(wid * CP, CP)])


def _topk(probs_t):
    k = functools.partial(
        pl.kernel,
        out_type=[
            jax.ShapeDtypeStruct((E * CP,), jnp.int32),
            jax.ShapeDtypeStruct((E * CP,), jnp.float32),
        ],
        mesh=_SC_MESH,
        scratch_types=[
            pltpu.VMEM((N,), jnp.float32),
            pltpu.VMEM((N,), jnp.int32),
            pltpu.VMEM((CP + LANES,), jnp.int32),
            pltpu.VMEM((CP + LANES,), jnp.float32),
        ],
        compiler_params=pltpu.CompilerParams(needs_layout_passes=False),
    )(_topk_kernel)
    return k(probs_t)


_GROWS = CP // 4
_GBATCH = 24
_GSTEPS = _GROWS // _GBATCH


def _gather_kernel(hid_hbm, idx_hbm, xg_hbm, idx_v, buf0, buf1, gs0, gs1, ws0, ws1):
    cid = lax.axis_index("c")
    sid = lax.axis_index("s")
    wid = sid * NC + cid
    e = wid // 4
    q = wid % 4
    base = q * _GROWS
    pltpu.sync_copy(idx_hbm.at[pl.ds(e * CP + base, _GROWS)], idx_v)
    bufs = (buf0, buf1)
    gsems = (gs0, gs1)
    wsems = (ws0, ws1)
    wdesc = [None, None]
    for b in range(_GSTEPS):
        s = b % 2
        if wdesc[s] is not None:
            wdesc[s].wait()
        g = pltpu.async_copy(
            hid_hbm.at[idx_v.at[pl.ds(b * _GBATCH, _GBATCH)]], bufs[s], gsems[s])
        g.wait()
        wdesc[s] = pltpu.async_copy(
            bufs[s], xg_hbm.at[e, pl.ds(base + b * _GBATCH, _GBATCH), :], wsems[s])
    for s in range(2):
        if wdesc[s] is not None:
            wdesc[s].wait()


def _gather(hidden_flat, idx):
    k = functools.partial(
        pl.kernel,
        out_type=jax.ShapeDtypeStruct((E, CP, D), jnp.float32),
        mesh=_SC_MESH,
        scratch_types=[
            pltpu.VMEM((_GROWS,), jnp.int32),
            pltpu.VMEM((_GBATCH, D), jnp.float32),
            pltpu.VMEM((_GBATCH, D), jnp.float32),
            pltpu.SemaphoreType.DMA,
            pltpu.SemaphoreType.DMA,
            pltpu.SemaphoreType.DMA,
            pltpu.SemaphoreType.DMA,
        ],
        compiler_params=pltpu.CompilerParams(needs_layout_passes=False),
    )(_gather_kernel)
    return k(hidden_flat, idx)


_SQRT_HALF = float(1.0 / math.sqrt(2.0))


def _ffn_body(xg_ref, w1_ref, w2_ref, ws_ref, out_ref):
    kf = pl.program_id(1)
    x = xg_ref[0].astype(jnp.bfloat16)
    h = lax.dot_general(x, w1_ref[0], (((1,), (1,)), ((), ())),
                        preferred_element_type=jnp.float32)
    h = 0.5 * h * (1.0 + lax.erf(h * _SQRT_HALF))
    hb = h.astype(jnp.bfloat16)
    part = lax.dot_general(hb, w2_ref[0], (((1,), (1,)), ((), ())),
                           preferred_element_type=jnp.float32)

    @pl.when(kf == 0)
    def _():
        out_ref[0] = part

    @pl.when(kf > 0)
    def _():
        out_ref[0] += part

    @pl.when(kf == KF - 1)
    def _():
        out_ref[0] *= ws_ref[0, 0][:, None]


def _ffn(xg, W1, W2, wsel):
    return pl.pallas_call(
        _ffn_body,
        grid=(E, KF),
        in_specs=[
            pl.BlockSpec((1, CP, D), lambda e, k: (e, 0, 0)),
            pl.BlockSpec((1, BF, D), lambda e, k: (e, k, 0)),
            pl.BlockSpec((1, D, BF), lambda e, k: (e, 0, k)),
            pl.BlockSpec((1, 1, CP), lambda e, k: (e, 0, 0)),
        ],
        out_specs=pl.BlockSpec((1, CP, D), lambda e, k: (e, 0, 0)),
        out_shape=jax.ShapeDtypeStruct((E, CP, D), jnp.float32),
        compiler_params=pltpu.CompilerParams(
            dimension_semantics=("arbitrary", "arbitrary")),
    )(xg, W1, W2, wsel.reshape(E, 1, CP))


_TPW = N // (NC * NS)
_NGRP = _TPW // LANES
_ZERO_ROW = CAP
_NENT = E * CP


def _combine_kernel(outg_hbm, idx_hbm, out_hbm,
                    idxf_v, cnt_v, cont_v, gi_v, acc_v, buf_v, gsem):
    cid = lax.axis_index("c")
    sid = lax.axis_index("s")
    wid = sid * NC + cid
    base = wid * _TPW
    pltpu.sync_copy(idx_hbm, idxf_v)
    lane_iota = jnp.arange(LANES, dtype=jnp.int32)
    zi = jnp.zeros((LANES,), jnp.int32)
    sent = jnp.full((LANES,), _ZERO_ROW, jnp.int32)

    def zfill(i, _):
        cnt_v[pl.ds(i * LANES, LANES)] = zi
        return 0
    lax.fori_loop(0, _TPW // LANES, zfill, 0)

    def sfill(i, _):
        cont_v[pl.ds(i * LANES, LANES)] = sent
        return 0
    lax.fori_loop(0, (E * _TPW) // LANES, sfill, 0)

    def scan(i, _):
        t = idxf_v[pl.ds(i * LANES, LANES)]
        r = lane_iota + i * LANES
        tl = t - base
        m = (tl >= 0) & (tl < _TPW)
        tls = jnp.where(m, tl, 0)
        c = plsc.load_gather(cnt_v, [tls], mask=m)
        c = jnp.where(m, c, 0)
        plsc.store_scatter(cont_v, [c * _TPW + tls], r, mask=m)
        plsc.store_scatter(cnt_v, [tls], c + 1, mask=m)
        return 0
    lax.fori_loop(0, _NENT // LANES, scan, 0)

    for g in range(_NGRP):
        cg = cnt_v[pl.ds(g * LANES, LANES)]
        mx = jnp.maximum(jnp.max(cg), 1)
        pltpu.sync_copy(cont_v.at[pl.ds(g * LANES, LANES)], gi_v)
        pltpu.async_copy(outg_hbm.at[gi_v], acc_v, gsem).wait()

        def rstep(j, _):
            pltpu.sync_copy(
                cont_v.at[pl.ds(j * _TPW + g * LANES, LANES)], gi_v)
            pltpu.async_copy(outg_hbm.at[gi_v], buf_v, gsem).wait()

            def radd(row, _):
                def cadd(kk, _):
                    col = kk * LANES
                    acc_v[row, pl.ds(col, LANES)] += buf_v[row, pl.ds(col, LANES)]
                    return 0
                lax.fori_loop(0, D // LANES, cadd, 0)
                return 0
            lax.fori_loop(0, LANES, radd, 0)
            return 0
        lax.fori_loop(1, mx, rstep, 0)
        pltpu.sync_copy(acc_v, out_hbm.at[pl.ds(base + g * LANES, LANES), :])


def _combine(outg_flat, idx):
    k = functools.partial(
        pl.kernel,
        out_type=jax.ShapeDtypeStruct((N, D), jnp.float32),
        mesh=_SC_MESH,
        scratch_types=[
            pltpu.VMEM((_NENT,), jnp.int32),
            pltpu.VMEM((_TPW,), jnp.int32),
            pltpu.VMEM((E * _TPW,), jnp.int32),
            pltpu.VMEM((LANES,), jnp.int32),
            pltpu.VMEM((LANES, D), jnp.float32),
            pltpu.VMEM((LANES, D), jnp.float32),
            pltpu.SemaphoreType.DMA,
        ],
        compiler_params=pltpu.CompilerParams(needs_layout_passes=False),
    )(_combine_kernel)
    return k(outg_flat, idx)


def kernel(hidden, Wr, W1, W2):
    hidden_flat = hidden.reshape(N, D)
    probs_t, ent_parts = _router(hidden_flat, Wr)
    idx, wsel = _topk(probs_t.reshape(E * N))
    xg = _gather(hidden_flat, idx)
    outg = _ffn(xg, W1, W2, wsel)
    out = jnp.zeros((N, D), jnp.float32).at[idx].add(outg.reshape(E * CP, D))
    aux_loss = jnp.sum(ent_parts) / N
    return (out.reshape(B, T, D), aux_loss)
